# baseline (speedup 1.0000x reference)
import functools

import jax
import jax.numpy as jnp
import numpy as np
from jax import lax
from jax.experimental import pallas as pl
from jax.experimental.pallas import tpu as pltpu
from jax.experimental.pallas import tpu_sc as plsc

N = 10000
E = 160000
HID = 128
GRAPHS = 64
LAYERS = 4
RADIUS = 1.0
CUT_P = 6.0

NC = 2
NS = 16
NW = NC * NS
CHUNK = 64
TPW = 81
EP = NW * TPW * CHUNK
NACC = N + 8
ROWS_PER_SUB = 624

BE = 1024
BN = 1000


def _gelu(v):
    c = np.sqrt(2.0 / np.pi).astype(np.float32)
    return 0.5 * v * (1.0 + jnp.tanh(c * (v + 0.044715 * (v * v * v))))


def _poly_perm():
    idx = list(range(3))
    off = 3
    for t in (2, 3, 4):
        n = 3 ** t
        for m in range(n):
            dig = []
            mm = m
            for _ in range(t):
                dig.append(mm % 3)
                mm //= 3
            rev = 0
            for d in dig:
                rev = rev * 3 + d
            idx.append(off + rev)
        off += n
    return np.array(idx, dtype=np.int32)


_PERM = _poly_perm()


def _embed_body(x_ref, wx_ref, h_ref):
    h_ref[...] = jnp.dot(x_ref[...], wx_ref[...],
                         preferred_element_type=jnp.float32)


def _edge_body(attr_ref, dists_ref, wb1_ref, bb1_ref, wb2_ref, bb2_ref,
               wk_ref, k0_ref, k1_ref, k2_ref, k3_ref):
    a = attr_ref[...]
    a0 = a[:, 0:1]
    a1 = a[:, 1:2]
    a2 = a[:, 2:3]
    f2 = jnp.concatenate([a * a0, a * a1, a * a2], axis=1)
    f3 = jnp.concatenate([f2 * a0, f2 * a1, f2 * a2], axis=1)
    f4 = jnp.concatenate([f3 * a0, f3 * a1, f3 * a2], axis=1)
    feats = jnp.concatenate([a, f2, f3, f4], axis=1)
    hb = _gelu(jnp.dot(feats, wb1_ref[...],
                       preferred_element_type=jnp.float32) + bb1_ref[0, :])
    basis = _gelu(jnp.dot(hb, wb2_ref[...],
                          preferred_element_type=jnp.float32) + bb2_ref[0, :])
    d = dists_ref[...]
    p = CUT_P
    r = d * (1.0 / RADIUS)
    r2 = r * r
    r3 = r2 * r
    r6 = r3 * r3
    r7 = r6 * r
    r8 = r7 * r
    env = (1.0 - ((p + 1.0) * (p + 2.0) / 2.0) * r6
           + p * (p + 2.0) * r7
           - (p * (p + 1.0) / 2.0) * r8)
    env = env * (d < RADIUS).astype(jnp.float32)
    kb = basis * env
    wk = wk_ref[...]
    k0_ref[...] = jnp.dot(kb, wk[:, 0:128],
                          preferred_element_type=jnp.float32)
    k1_ref[...] = jnp.dot(kb, wk[:, 128:256],
                          preferred_element_type=jnp.float32)
    k2_ref[...] = jnp.dot(kb, wk[:, 256:384],
                          preferred_element_type=jnp.float32)
    k3_ref[...] = jnp.dot(kb, wk[:, 384:512],
                          preferred_element_type=jnp.float32)


def _sc_body(h_hbm, k_hbm, src_hbm, dst_hbm, zeros_hbm, out_hbm,
             idx_s, idx_d, rows, kbuf, acc,
             sem_is, sem_id, sem_g, sem_k, sem_s):
    c = lax.axis_index("c")
    s = lax.axis_index("s")
    w = c * NS + s

    pltpu.sync_copy(zeros_hbm.at[pl.ds(s * ROWS_PER_SUB, ROWS_PER_SUB), :],
                    acc.at[pl.ds(s * ROWS_PER_SUB, ROWS_PER_SUB), :])

    @pl.when(s == NS - 1)
    def _():
        tail = NS * ROWS_PER_SUB
        pltpu.sync_copy(zeros_hbm.at[pl.ds(tail, NACC - tail), :],
                        acc.at[pl.ds(tail, NACC - tail), :])

    plsc.subcore_barrier()

    def base_of(t):
        tt = jnp.minimum(t, TPW - 1)
        return (tt * NW + w) * CHUNK

    def issue_idx(t, b):
        pltpu.async_copy(src_hbm.at[pl.ds(base_of(t), CHUNK)],
                         idx_s[b], sem_is[b])
        pltpu.async_copy(dst_hbm.at[pl.ds(base_of(t), CHUNK)],
                         idx_d[b], sem_id[b])

    def wait_idx(t, b):
        pltpu.make_async_copy(src_hbm.at[pl.ds(base_of(t), CHUNK)],
                              idx_s[b], sem_is[b]).wait()
        pltpu.make_async_copy(dst_hbm.at[pl.ds(base_of(t), CHUNK)],
                              idx_d[b], sem_id[b]).wait()

    def issue_load(t, b):
        pltpu.async_copy(h_hbm.at[idx_s[b]], rows[b], sem_g[b])
        pltpu.async_copy(k_hbm.at[pl.ds(base_of(t), CHUNK), :],
                         kbuf[b], sem_k[b])

    def wait_load(t, b):
        pltpu.make_async_copy(h_hbm.at[idx_s[b]], rows[b], sem_g[b]).wait()
        pltpu.make_async_copy(k_hbm.at[pl.ds(base_of(t), CHUNK), :],
                              kbuf[b], sem_k[b]).wait()

    def issue_scatter(b):
        pltpu.async_copy(rows[b], acc.at[idx_d[b]], sem_s[b], add=True)

    def wait_scatter(b):
        pltpu.make_async_copy(rows[b], acc.at[idx_d[b]], sem_s[b]).wait()

    pltpu.sync_copy(src_hbm.at[pl.ds(base_of(0), CHUNK)], idx_s[0])
    pltpu.sync_copy(dst_hbm.at[pl.ds(base_of(0), CHUNK)], idx_d[0])
    issue_load(0, 0)
    issue_idx(1, 1)

    def group_body(g, carry):
        for b in range(3):
            t = g * 3 + b
            b1 = (b + 1) % 3
            b2 = (b + 2) % 3
            wait_idx(t + 1, b1)
            issue_load(t + 1, b1)
            wait_load(t, b)

            def mul_body(i, carry2):
                for v in range(2):
                    for u in range(8):
                        sl = pl.ds(u * 16, 16)
                        rows[b][2 * i + v, sl] = (rows[b][2 * i + v, sl]
                                                  * kbuf[b][2 * i + v, sl])
                return carry2

            lax.fori_loop(0, CHUNK // 2, mul_body, 0)
            if b == 0:
                @pl.when(g > 0)
                def _():
                    wait_scatter(b2)
            else:
                wait_scatter(b2)
            issue_scatter(b)
            issue_idx(t + 2, b2)
        return carry

    lax.fori_loop(0, TPW // 3, group_body, 0)

    wait_scatter((TPW - 1) % 3)
    wait_load(TPW - 1, TPW % 3)
    wait_idx(TPW - 1, (TPW + 1) % 3)

    plsc.subcore_barrier()
    pltpu.sync_copy(acc.at[pl.ds(s * ROWS_PER_SUB, ROWS_PER_SUB), :],
                    out_hbm.at[c, pl.ds(s * ROWS_PER_SUB, ROWS_PER_SUB), :])

    @pl.when(s == NS - 1)
    def _():
        tail = NS * ROWS_PER_SUB
        pltpu.sync_copy(acc.at[pl.ds(tail, N - tail), :],
                        out_hbm.at[c, pl.ds(tail, N - tail), :])


def _node_body(part_ref, h_ref, batch_ref, bk_ref, g_ref, be_ref,
               w1_ref, b1_ref, w2_ref, b2_ref, ls_ref, wr_ref, br_ref,
               hout_ref, pool_ref):
    i = pl.program_id(0)
    agg = part_ref[0] + part_ref[1] + bk_ref[0, :]
    m = jnp.mean(agg, axis=-1, keepdims=True)
    ctr = agg - m
    v = jnp.mean(ctr * ctr, axis=-1, keepdims=True)
    y = ctr * jax.lax.rsqrt(v + 1e-5) * g_ref[0, :] + be_ref[0, :]
    y = _gelu(jnp.dot(y, w1_ref[...],
                      preferred_element_type=jnp.float32) + b1_ref[0, :])
    y = jnp.dot(y, w2_ref[...],
                preferred_element_type=jnp.float32) + b2_ref[0, :]
    hn = h_ref[...] + ls_ref[0, :] * y
    hout_ref[...] = hn
    r = jnp.dot(hn, wr_ref[...],
                preferred_element_type=jnp.float32) + br_ref[0, 0]
    b = batch_ref[0]
    gid = jax.lax.broadcasted_iota(jnp.int32, (GRAPHS, BN), 0)
    oh = (gid == b).astype(jnp.float32)
    pr = jnp.dot(oh, r, preferred_element_type=jnp.float32)

    @pl.when(i == 0)
    def _():
        pool_ref[...] = pr

    @pl.when(i != 0)
    def _():
        pool_ref[...] = pool_ref[...] + pr


def _const_spec(shape):
    return pl.BlockSpec(shape, lambda i: (0,) * len(shape))


@jax.jit
def kernel(x, attr, dists, edge_index, batch, params):
    pad = EP - E
    src = jnp.concatenate([edge_index[0], jnp.zeros((pad,), jnp.int32)])
    dst = jnp.concatenate([edge_index[1], jnp.full((pad,), N, jnp.int32)])
    attr_p = jnp.concatenate([attr, jnp.zeros((pad, 3), jnp.float32)])
    dists_p = jnp.concatenate([dists, jnp.zeros((pad, 1), jnp.float32)])
    zeros = jnp.zeros((NACC, HID), jnp.float32)
    batch3 = batch.reshape(N // BN, 1, BN)

    h = pl.pallas_call(
        _embed_body,
        grid=(N // BN,),
        in_specs=[pl.BlockSpec((BN, 128), lambda i: (i, 0)),
                  _const_spec((128, HID))],
        out_specs=pl.BlockSpec((BN, HID), lambda i: (i, 0)),
        out_shape=jax.ShapeDtypeStruct((N, HID), jnp.float32),
    )(x, params['Wx'])

    wb1 = params['Wb1'][_PERM, :]
    wk = jnp.concatenate([params['Wk%d' % i] for i in range(LAYERS)], axis=1)
    kspec = pl.BlockSpec((BE, HID), lambda i: (i, 0))
    ks = pl.pallas_call(
        _edge_body,
        grid=(EP // BE,),
        in_specs=[pl.BlockSpec((BE, 3), lambda i: (i, 0)),
                  pl.BlockSpec((BE, 1), lambda i: (i, 0)),
                  _const_spec((120, HID)),
                  _const_spec((1, HID)),
                  _const_spec((HID, HID)),
                  _const_spec((1, HID)),
                  _const_spec((HID, 4 * HID))],
        out_specs=[kspec, kspec, kspec, kspec],
        out_shape=[jax.ShapeDtypeStruct((EP, HID), jnp.float32)] * 4,
    )(attr_p, dists_p, wb1, params['bb1'].reshape(1, HID),
      params['Wb2'], params['bb2'].reshape(1, HID), wk)

    sc_call = pl.kernel(
        _sc_body,
        out_type=jax.ShapeDtypeStruct((NC, N, HID), jnp.float32),
        mesh=plsc.VectorSubcoreMesh(core_axis_name="c", subcore_axis_name="s",
                                    num_cores=NC, num_subcores=NS),
        scratch_types=[
            [pltpu.VMEM((CHUNK,), jnp.int32) for _ in range(3)],
            [pltpu.VMEM((CHUNK,), jnp.int32) for _ in range(3)],
            [pltpu.VMEM((CHUNK, HID), jnp.float32) for _ in range(3)],
            [pltpu.VMEM((CHUNK, HID), jnp.float32) for _ in range(3)],
            pltpu.VMEM_SHARED((NACC, HID), jnp.float32),
            [pltpu.SemaphoreType.DMA for _ in range(3)],
            [pltpu.SemaphoreType.DMA for _ in range(3)],
            [pltpu.SemaphoreType.DMA for _ in range(3)],
            [pltpu.SemaphoreType.DMA for _ in range(3)],
            [pltpu.SemaphoreType.DMA for _ in range(3)],
        ],
    )

    node_call = pl.pallas_call(
        _node_body,
        grid=(N // BN,),
        in_specs=[pl.BlockSpec((NC, BN, HID), lambda i: (0, i, 0)),
                  pl.BlockSpec((BN, HID), lambda i: (i, 0)),
                  pl.BlockSpec((1, 1, BN), lambda i: (i, 0, 0)),
                  _const_spec((1, HID)),
                  _const_spec((1, HID)),
                  _const_spec((1, HID)),
                  _const_spec((HID, 4 * HID)),
                  _const_spec((1, 4 * HID)),
                  _const_spec((4 * HID, HID)),
                  _const_spec((1, HID)),
                  _const_spec((1, HID)),
                  _const_spec((HID, 1)),
                  _const_spec((1, 1))],
        out_specs=[pl.BlockSpec((BN, HID), lambda i: (i, 0)),
                   pl.BlockSpec((GRAPHS, 1), lambda i: (0, 0))],
        out_shape=[jax.ShapeDtypeStruct((N, HID), jnp.float32),
                   jax.ShapeDtypeStruct((GRAPHS, 1), jnp.float32)],
    )

    pooled = None
    for i in range(LAYERS):
        part = sc_call(h, ks[i], src, dst, zeros)
        h, pr = node_call(
            part, h, batch3,
            params['bk%d' % i].reshape(1, HID),
            params['g%d' % i].reshape(1, HID),
            params['be%d' % i].reshape(1, HID),
            params['W1_%d' % i],
            params['b1_%d' % i].reshape(1, 4 * HID),
            params['W2_%d' % i],
            params['b2_%d' % i].reshape(1, HID),
            params['ls%d' % i].reshape(1, HID),
            params['Wr%d' % i],
            params['br%d' % i].reshape(1, 1))
        pooled = pr if pooled is None else pooled + pr

    return pooled * (1.0 / LAYERS)


# speedup vs baseline: 1.0566x; 1.0566x over previous
import functools

import jax
import jax.numpy as jnp
import numpy as np
from jax import lax
from jax.experimental import pallas as pl
from jax.experimental.pallas import tpu as pltpu
from jax.experimental.pallas import tpu_sc as plsc

N = 10000
E = 160000
HID = 128
GRAPHS = 64
LAYERS = 4
RADIUS = 1.0
CUT_P = 6.0

NC = 2
NS = 16
NW = NC * NS
EW = E // NW
CHUNK = 40
NCHUNK = EW // CHUNK
ROWS_PER_SUB = 624

BE = 1000
BN = 1000


def _gelu(v):
    c = np.sqrt(2.0 / np.pi).astype(np.float32)
    return 0.5 * v * (1.0 + jnp.tanh(c * (v + 0.044715 * (v * v * v))))


def _poly_perm():
    idx = list(range(3))
    off = 3
    for t in (2, 3, 4):
        n = 3 ** t
        for m in range(n):
            dig = []
            mm = m
            for _ in range(t):
                dig.append(mm % 3)
                mm //= 3
            rev = 0
            for d in dig:
                rev = rev * 3 + d
            idx.append(off + rev)
        off += n
    return np.array(idx, dtype=np.int32)


_PERM = _poly_perm()


def _embed_body(x_ref, wx_ref, h_ref):
    h_ref[...] = jnp.dot(x_ref[...], wx_ref[...],
                         preferred_element_type=jnp.float32)


def _edge_body(attr_ref, dists_ref, wb1_ref, bb1_ref, wb2_ref, bb2_ref,
               wk_ref, k0_ref, k1_ref, k2_ref, k3_ref):
    a = attr_ref[...]
    a0 = a[:, 0:1]
    a1 = a[:, 1:2]
    a2 = a[:, 2:3]
    f2 = jnp.concatenate([a * a0, a * a1, a * a2], axis=1)
    f3 = jnp.concatenate([f2 * a0, f2 * a1, f2 * a2], axis=1)
    f4 = jnp.concatenate([f3 * a0, f3 * a1, f3 * a2], axis=1)
    feats = jnp.concatenate([a, f2, f3, f4], axis=1)
    hb = _gelu(jnp.dot(feats, wb1_ref[...],
                       preferred_element_type=jnp.float32) + bb1_ref[0, :])
    basis = _gelu(jnp.dot(hb, wb2_ref[...],
                          preferred_element_type=jnp.float32) + bb2_ref[0, :])
    d = dists_ref[...]
    p = CUT_P
    r = d * (1.0 / RADIUS)
    r2 = r * r
    r3 = r2 * r
    r6 = r3 * r3
    r7 = r6 * r
    r8 = r7 * r
    env = (1.0 - ((p + 1.0) * (p + 2.0) / 2.0) * r6
           + p * (p + 2.0) * r7
           - (p * (p + 1.0) / 2.0) * r8)
    env = env * (d < RADIUS).astype(jnp.float32)
    kb = basis * env
    wk = wk_ref[...]
    k0_ref[...] = jnp.dot(kb, wk[:, 0:128],
                          preferred_element_type=jnp.float32)
    k1_ref[...] = jnp.dot(kb, wk[:, 128:256],
                          preferred_element_type=jnp.float32)
    k2_ref[...] = jnp.dot(kb, wk[:, 256:384],
                          preferred_element_type=jnp.float32)
    k3_ref[...] = jnp.dot(kb, wk[:, 384:512],
                          preferred_element_type=jnp.float32)


def _sc_body(h_hbm, k_hbm, src_hbm, dst_hbm, zeros_hbm, out_hbm,
             idx_s, idx_d, rows, kbuf, acc, sem):
    c = lax.axis_index("c")
    s = lax.axis_index("s")
    w = c * NS + s

    pltpu.sync_copy(zeros_hbm.at[pl.ds(s * ROWS_PER_SUB, ROWS_PER_SUB), :],
                    acc.at[pl.ds(s * ROWS_PER_SUB, ROWS_PER_SUB), :])

    @pl.when(s == NS - 1)
    def _():
        tail = NS * ROWS_PER_SUB
        pltpu.sync_copy(zeros_hbm.at[pl.ds(tail, N - tail), :],
                        acc.at[pl.ds(tail, N - tail), :])

    plsc.subcore_barrier()

    def chunk_body(j, carry):
        base = w * EW + j * CHUNK
        pltpu.sync_copy(src_hbm.at[pl.ds(base, CHUNK)], idx_s)
        pltpu.sync_copy(dst_hbm.at[pl.ds(base, CHUNK)], idx_d)
        pltpu.async_copy(h_hbm.at[idx_s], rows, sem).wait()
        pltpu.sync_copy(k_hbm.at[pl.ds(base, CHUNK), :], kbuf)

        def mul_body(i, carry2):
            for u in range(8):
                sl = pl.ds(u * 16, 16)
                rows[i, sl] = rows[i, sl] * kbuf[i, sl]
            return carry2

        lax.fori_loop(0, CHUNK, mul_body, 0)
        pltpu.sync_copy(rows, acc.at[idx_d], add=True)
        return carry

    lax.fori_loop(0, NCHUNK, chunk_body, 0)

    plsc.subcore_barrier()
    pltpu.sync_copy(acc.at[pl.ds(s * ROWS_PER_SUB, ROWS_PER_SUB), :],
                    out_hbm.at[c, pl.ds(s * ROWS_PER_SUB, ROWS_PER_SUB), :])

    @pl.when(s == NS - 1)
    def _():
        tail = NS * ROWS_PER_SUB
        pltpu.sync_copy(acc.at[pl.ds(tail, N - tail), :],
                        out_hbm.at[c, pl.ds(tail, N - tail), :])


def _node_body(part_ref, h_ref, batch_ref, bk_ref, g_ref, be_ref,
               w1_ref, b1_ref, w2_ref, b2_ref, ls_ref, wr_ref, br_ref,
               hout_ref, pool_ref):
    i = pl.program_id(0)
    agg = part_ref[0] + part_ref[1] + bk_ref[0, :]
    m = jnp.mean(agg, axis=-1, keepdims=True)
    ctr = agg - m
    v = jnp.mean(ctr * ctr, axis=-1, keepdims=True)
    y = ctr * jax.lax.rsqrt(v + 1e-5) * g_ref[0, :] + be_ref[0, :]
    y = _gelu(jnp.dot(y, w1_ref[...],
                      preferred_element_type=jnp.float32) + b1_ref[0, :])
    y = jnp.dot(y, w2_ref[...],
                preferred_element_type=jnp.float32) + b2_ref[0, :]
    hn = h_ref[...] + ls_ref[0, :] * y
    hout_ref[...] = hn
    r = jnp.dot(hn, wr_ref[...],
                preferred_element_type=jnp.float32) + br_ref[0, 0]
    b = batch_ref[0]
    gid = jax.lax.broadcasted_iota(jnp.int32, (GRAPHS, BN), 0)
    oh = (gid == b).astype(jnp.float32)
    pr = jnp.dot(oh, r, preferred_element_type=jnp.float32)

    @pl.when(i == 0)
    def _():
        pool_ref[...] = pr

    @pl.when(i != 0)
    def _():
        pool_ref[...] = pool_ref[...] + pr


def _const_spec(shape):
    return pl.BlockSpec(shape, lambda i: (0,) * len(shape))


@jax.jit
def kernel(x, attr, dists, edge_index, batch, params):
    src = edge_index[0]
    dst = edge_index[1]
    zeros = jnp.zeros((N, HID), jnp.float32)
    batch3 = batch.reshape(N // BN, 1, BN)

    h = pl.pallas_call(
        _embed_body,
        grid=(N // BN,),
        in_specs=[pl.BlockSpec((BN, 128), lambda i: (i, 0)),
                  _const_spec((128, HID))],
        out_specs=pl.BlockSpec((BN, HID), lambda i: (i, 0)),
        out_shape=jax.ShapeDtypeStruct((N, HID), jnp.float32),
    )(x, params['Wx'])

    wb1 = params['Wb1'][_PERM, :]
    wk = jnp.concatenate([params['Wk%d' % i] for i in range(LAYERS)], axis=1)
    kspec = pl.BlockSpec((BE, HID), lambda i: (i, 0))
    ks = pl.pallas_call(
        _edge_body,
        grid=(E // BE,),
        in_specs=[pl.BlockSpec((BE, 3), lambda i: (i, 0)),
                  pl.BlockSpec((BE, 1), lambda i: (i, 0)),
                  _const_spec((120, HID)),
                  _const_spec((1, HID)),
                  _const_spec((HID, HID)),
                  _const_spec((1, HID)),
                  _const_spec((HID, 4 * HID))],
        out_specs=[kspec, kspec, kspec, kspec],
        out_shape=[jax.ShapeDtypeStruct((E, HID), jnp.float32)] * 4,
    )(attr, dists, wb1, params['bb1'].reshape(1, HID),
      params['Wb2'], params['bb2'].reshape(1, HID), wk)

    sc_call = pl.kernel(
        _sc_body,
        out_type=jax.ShapeDtypeStruct((NC, N, HID), jnp.float32),
        mesh=plsc.VectorSubcoreMesh(core_axis_name="c", subcore_axis_name="s",
                                    num_cores=NC, num_subcores=NS),
        scratch_types=[
            pltpu.VMEM((CHUNK,), jnp.int32),
            pltpu.VMEM((CHUNK,), jnp.int32),
            pltpu.VMEM((CHUNK, HID), jnp.float32),
            pltpu.VMEM((CHUNK, HID), jnp.float32),
            pltpu.VMEM_SHARED((N, HID), jnp.float32),
            pltpu.SemaphoreType.DMA,
        ],
    )

    node_call = pl.pallas_call(
        _node_body,
        grid=(N // BN,),
        in_specs=[pl.BlockSpec((NC, BN, HID), lambda i: (0, i, 0)),
                  pl.BlockSpec((BN, HID), lambda i: (i, 0)),
                  pl.BlockSpec((1, 1, BN), lambda i: (i, 0, 0)),
                  _const_spec((1, HID)),
                  _const_spec((1, HID)),
                  _const_spec((1, HID)),
                  _const_spec((HID, 4 * HID)),
                  _const_spec((1, 4 * HID)),
                  _const_spec((4 * HID, HID)),
                  _const_spec((1, HID)),
                  _const_spec((1, HID)),
                  _const_spec((HID, 1)),
                  _const_spec((1, 1))],
        out_specs=[pl.BlockSpec((BN, HID), lambda i: (i, 0)),
                   pl.BlockSpec((GRAPHS, 1), lambda i: (0, 0))],
        out_shape=[jax.ShapeDtypeStruct((N, HID), jnp.float32),
                   jax.ShapeDtypeStruct((GRAPHS, 1), jnp.float32)],
    )

    pooled = None
    for i in range(LAYERS):
        part = sc_call(h, ks[i], src, dst, zeros)
        h, pr = node_call(
            part, h, batch3,
            params['bk%d' % i].reshape(1, HID),
            params['g%d' % i].reshape(1, HID),
            params['be%d' % i].reshape(1, HID),
            params['W1_%d' % i],
            params['b1_%d' % i].reshape(1, 4 * HID),
            params['W2_%d' % i],
            params['b2_%d' % i].reshape(1, HID),
            params['ls%d' % i].reshape(1, HID),
            params['Wr%d' % i],
            params['br%d' % i].reshape(1, 1))
        pooled = pr if pooled is None else pooled + pr

    return pooled * (1.0 / LAYERS)


# speedup vs baseline: 1.2977x; 1.2281x over previous
import functools

import jax
import jax.numpy as jnp
import numpy as np
from jax import lax
from jax.experimental import pallas as pl
from jax.experimental.pallas import tpu as pltpu
from jax.experimental.pallas import tpu_sc as plsc

N = 10000
E = 160000
HID = 128
GRAPHS = 64
LAYERS = 4
RADIUS = 1.0
CUT_P = 6.0

NC = 2
NS = 16
NW = NC * NS
EW = E // NW
CHUNK = 40
NCHUNK = EW // CHUNK
ROWS_PER_SUB = 624

BE = 1000
BN = 1000


def _gelu(v):
    c = np.sqrt(2.0 / np.pi).astype(np.float32)
    return 0.5 * v * (1.0 + jnp.tanh(c * (v + 0.044715 * (v * v * v))))


def _poly_perm():
    idx = list(range(3))
    off = 3
    for t in (2, 3, 4):
        n = 3 ** t
        for m in range(n):
            dig = []
            mm = m
            for _ in range(t):
                dig.append(mm % 3)
                mm //= 3
            rev = 0
            for d in dig:
                rev = rev * 3 + d
            idx.append(off + rev)
        off += n
    return np.array(idx, dtype=np.int32)


_PERM = _poly_perm()


def _embed_body(x_ref, wx_ref, h_ref):
    h_ref[...] = jnp.dot(x_ref[...], wx_ref[...],
                         preferred_element_type=jnp.float32)


def _edge_body(attr_ref, dists_ref, wb1_ref, bb1_ref, wb2_ref, bb2_ref,
               wk_ref, k0_ref, k1_ref, k2_ref, k3_ref):
    a = attr_ref[...]
    a0 = a[:, 0:1]
    a1 = a[:, 1:2]
    a2 = a[:, 2:3]
    f2 = jnp.concatenate([a * a0, a * a1, a * a2], axis=1)
    f3 = jnp.concatenate([f2 * a0, f2 * a1, f2 * a2], axis=1)
    f4 = jnp.concatenate([f3 * a0, f3 * a1, f3 * a2], axis=1)
    feats = jnp.concatenate([a, f2, f3, f4], axis=1)
    hb = _gelu(jnp.dot(feats, wb1_ref[...],
                       preferred_element_type=jnp.float32) + bb1_ref[0, :])
    basis = _gelu(jnp.dot(hb, wb2_ref[...],
                          preferred_element_type=jnp.float32) + bb2_ref[0, :])
    d = dists_ref[...]
    p = CUT_P
    r = d * (1.0 / RADIUS)
    r2 = r * r
    r3 = r2 * r
    r6 = r3 * r3
    r7 = r6 * r
    r8 = r7 * r
    env = (1.0 - ((p + 1.0) * (p + 2.0) / 2.0) * r6
           + p * (p + 2.0) * r7
           - (p * (p + 1.0) / 2.0) * r8)
    env = env * (d < RADIUS).astype(jnp.float32)
    kb = basis * env
    wk = wk_ref[...]
    k0_ref[...] = jnp.dot(kb, wk[:, 0:128],
                          preferred_element_type=jnp.float32)
    k1_ref[...] = jnp.dot(kb, wk[:, 128:256],
                          preferred_element_type=jnp.float32)
    k2_ref[...] = jnp.dot(kb, wk[:, 256:384],
                          preferred_element_type=jnp.float32)
    k3_ref[...] = jnp.dot(kb, wk[:, 384:512],
                          preferred_element_type=jnp.float32)


def _sc_body(h_hbm, k_hbm, src_hbm, dst_hbm, zeros_hbm, out_hbm,
             idx_s, idx_d, rows, kbuf, acc, sem):
    c = lax.axis_index("c")
    s = lax.axis_index("s")
    w = c * NS + s

    pltpu.sync_copy(zeros_hbm.at[pl.ds(s * ROWS_PER_SUB, ROWS_PER_SUB), :],
                    acc.at[pl.ds(s * ROWS_PER_SUB, ROWS_PER_SUB), :])

    @pl.when(s == NS - 1)
    def _():
        tail = NS * ROWS_PER_SUB
        pltpu.sync_copy(zeros_hbm.at[pl.ds(tail, N - tail), :],
                        acc.at[pl.ds(tail, N - tail), :])

    plsc.subcore_barrier()

    def chunk_body(j, carry):
        base = w * EW + j * CHUNK
        pltpu.sync_copy(src_hbm.at[pl.ds(base, CHUNK)], idx_s)
        gd = pltpu.async_copy(h_hbm.at[idx_s], rows, sem)
        pltpu.sync_copy(dst_hbm.at[pl.ds(base, CHUNK)], idx_d)
        pltpu.sync_copy(k_hbm.at[pl.ds(base, CHUNK), :], kbuf)
        gd.wait()

        def mul_body(i, carry2):
            for u in range(8):
                sl = pl.ds(u * 16, 16)
                rows[i, sl] = rows[i, sl] * kbuf[i, sl]
            return carry2

        lax.fori_loop(0, CHUNK, mul_body, 0)
        pltpu.sync_copy(rows, acc.at[idx_d], add=True)
        return carry

    lax.fori_loop(0, NCHUNK, chunk_body, 0)

    plsc.subcore_barrier()
    pltpu.sync_copy(acc.at[pl.ds(s * ROWS_PER_SUB, ROWS_PER_SUB), :],
                    out_hbm.at[c, pl.ds(s * ROWS_PER_SUB, ROWS_PER_SUB), :])

    @pl.when(s == NS - 1)
    def _():
        tail = NS * ROWS_PER_SUB
        pltpu.sync_copy(acc.at[pl.ds(tail, N - tail), :],
                        out_hbm.at[c, pl.ds(tail, N - tail), :])


def _node_body(part_ref, h_ref, batch_ref, bk_ref, g_ref, be_ref,
               w1_ref, b1_ref, w2_ref, b2_ref, ls_ref, wr_ref, br_ref,
               hout_ref, pool_ref):
    i = pl.program_id(0)
    agg = part_ref[0] + part_ref[1] + bk_ref[0, :]
    m = jnp.mean(agg, axis=-1, keepdims=True)
    ctr = agg - m
    v = jnp.mean(ctr * ctr, axis=-1, keepdims=True)
    y = ctr * jax.lax.rsqrt(v + 1e-5) * g_ref[0, :] + be_ref[0, :]
    y = _gelu(jnp.dot(y, w1_ref[...],
                      preferred_element_type=jnp.float32) + b1_ref[0, :])
    y = jnp.dot(y, w2_ref[...],
                preferred_element_type=jnp.float32) + b2_ref[0, :]
    hn = h_ref[...] + ls_ref[0, :] * y
    hout_ref[...] = hn
    r = jnp.dot(hn, wr_ref[...],
                preferred_element_type=jnp.float32) + br_ref[0, 0]
    b = batch_ref[0]
    gid = jax.lax.broadcasted_iota(jnp.int32, (GRAPHS, BN), 0)
    oh = (gid == b).astype(jnp.float32)
    pr = jnp.dot(oh, r, preferred_element_type=jnp.float32)

    @pl.when(i == 0)
    def _():
        pool_ref[...] = pr

    @pl.when(i != 0)
    def _():
        pool_ref[...] = pool_ref[...] + pr


def _const_spec(shape):
    return pl.BlockSpec(shape, lambda i: (0,) * len(shape))


@jax.jit
def kernel(x, attr, dists, edge_index, batch, params):
    src = edge_index[0]
    dst = edge_index[1]
    zeros = jnp.zeros((N, HID), jnp.float32)
    batch3 = batch.reshape(N // BN, 1, BN)

    h = pl.pallas_call(
        _embed_body,
        grid=(N // BN,),
        in_specs=[pl.BlockSpec((BN, 128), lambda i: (i, 0)),
                  _const_spec((128, HID))],
        out_specs=pl.BlockSpec((BN, HID), lambda i: (i, 0)),
        out_shape=jax.ShapeDtypeStruct((N, HID), jnp.float32),
    )(x, params['Wx'])

    wb1 = params['Wb1'][_PERM, :]
    wk = jnp.concatenate([params['Wk%d' % i] for i in range(LAYERS)], axis=1)
    kspec = pl.BlockSpec((BE, HID), lambda i: (i, 0))
    ks = pl.pallas_call(
        _edge_body,
        grid=(E // BE,),
        in_specs=[pl.BlockSpec((BE, 3), lambda i: (i, 0)),
                  pl.BlockSpec((BE, 1), lambda i: (i, 0)),
                  _const_spec((120, HID)),
                  _const_spec((1, HID)),
                  _const_spec((HID, HID)),
                  _const_spec((1, HID)),
                  _const_spec((HID, 4 * HID))],
        out_specs=[kspec, kspec, kspec, kspec],
        out_shape=[jax.ShapeDtypeStruct((E, HID), jnp.float32)] * 4,
    )(attr, dists, wb1, params['bb1'].reshape(1, HID),
      params['Wb2'], params['bb2'].reshape(1, HID), wk)

    sc_call = pl.kernel(
        _sc_body,
        out_type=jax.ShapeDtypeStruct((NC, N, HID), jnp.float32),
        mesh=plsc.VectorSubcoreMesh(core_axis_name="c", subcore_axis_name="s",
                                    num_cores=NC, num_subcores=NS),
        scratch_types=[
            pltpu.VMEM((CHUNK,), jnp.int32),
            pltpu.VMEM((CHUNK,), jnp.int32),
            pltpu.VMEM((CHUNK, HID), jnp.float32),
            pltpu.VMEM((CHUNK, HID), jnp.float32),
            pltpu.VMEM_SHARED((N, HID), jnp.float32),
            pltpu.SemaphoreType.DMA,
        ],
    )

    node_call = pl.pallas_call(
        _node_body,
        grid=(N // BN,),
        in_specs=[pl.BlockSpec((NC, BN, HID), lambda i: (0, i, 0)),
                  pl.BlockSpec((BN, HID), lambda i: (i, 0)),
                  pl.BlockSpec((1, 1, BN), lambda i: (i, 0, 0)),
                  _const_spec((1, HID)),
                  _const_spec((1, HID)),
                  _const_spec((1, HID)),
                  _const_spec((HID, 4 * HID)),
                  _const_spec((1, 4 * HID)),
                  _const_spec((4 * HID, HID)),
                  _const_spec((1, HID)),
                  _const_spec((1, HID)),
                  _const_spec((HID, 1)),
                  _const_spec((1, 1))],
        out_specs=[pl.BlockSpec((BN, HID), lambda i: (i, 0)),
                   pl.BlockSpec((GRAPHS, 1), lambda i: (0, 0))],
        out_shape=[jax.ShapeDtypeStruct((N, HID), jnp.float32),
                   jax.ShapeDtypeStruct((GRAPHS, 1), jnp.float32)],
    )

    pooled = None
    for i in range(LAYERS):
        part = sc_call(h, ks[i], src, dst, zeros)
        h, pr = node_call(
            part, h, batch3,
            params['bk%d' % i].reshape(1, HID),
            params['g%d' % i].reshape(1, HID),
            params['be%d' % i].reshape(1, HID),
            params['W1_%d' % i],
            params['b1_%d' % i].reshape(1, 4 * HID),
            params['W2_%d' % i],
            params['b2_%d' % i].reshape(1, HID),
            params['ls%d' % i].reshape(1, HID),
            params['Wr%d' % i],
            params['br%d' % i].reshape(1, 1))
        pooled = pr if pooled is None else pooled + pr

    return pooled * (1.0 / LAYERS)


# speedup vs baseline: 1.3877x; 1.0694x over previous
import functools

import jax
import jax.numpy as jnp
import numpy as np
from jax import lax
from jax.experimental import pallas as pl
from jax.experimental.pallas import tpu as pltpu
from jax.experimental.pallas import tpu_sc as plsc

N = 10000
E = 160000
HID = 128
GRAPHS = 64
LAYERS = 4
RADIUS = 1.0
CUT_P = 6.0

NC = 2
NS = 16
NW = NC * NS
EW = E // NW
CHUNK = 40
NCHUNK = EW // CHUNK
ROWS_PER_SUB = 624

BE = 1000
BN = 1000


def _gelu(v):
    c = np.sqrt(2.0 / np.pi).astype(np.float32)
    return 0.5 * v * (1.0 + jnp.tanh(c * (v + 0.044715 * (v * v * v))))


def _poly_perm():
    idx = list(range(3))
    off = 3
    for t in (2, 3, 4):
        n = 3 ** t
        for m in range(n):
            dig = []
            mm = m
            for _ in range(t):
                dig.append(mm % 3)
                mm //= 3
            rev = 0
            for d in dig:
                rev = rev * 3 + d
            idx.append(off + rev)
        off += n
    return np.array(idx, dtype=np.int32)


_PERM = _poly_perm()


def _embed_body(x_ref, wx_ref, h_ref):
    h_ref[...] = jnp.dot(x_ref[...], wx_ref[...],
                         preferred_element_type=jnp.float32)


def _edge_body(attr_ref, dists_ref, wb1_ref, bb1_ref, wb2_ref, bb2_ref,
               wk_ref, k0_ref, k1_ref, k2_ref, k3_ref):
    a = attr_ref[...]
    a0 = a[:, 0:1]
    a1 = a[:, 1:2]
    a2 = a[:, 2:3]
    f2 = jnp.concatenate([a * a0, a * a1, a * a2], axis=1)
    f3 = jnp.concatenate([f2 * a0, f2 * a1, f2 * a2], axis=1)
    f4 = jnp.concatenate([f3 * a0, f3 * a1, f3 * a2], axis=1)
    feats = jnp.concatenate([a, f2, f3, f4], axis=1)
    hb = _gelu(jnp.dot(feats, wb1_ref[...],
                       preferred_element_type=jnp.float32) + bb1_ref[0, :])
    basis = _gelu(jnp.dot(hb, wb2_ref[...],
                          preferred_element_type=jnp.float32) + bb2_ref[0, :])
    d = dists_ref[...]
    p = CUT_P
    r = d * (1.0 / RADIUS)
    r2 = r * r
    r3 = r2 * r
    r6 = r3 * r3
    r7 = r6 * r
    r8 = r7 * r
    env = (1.0 - ((p + 1.0) * (p + 2.0) / 2.0) * r6
           + p * (p + 2.0) * r7
           - (p * (p + 1.0) / 2.0) * r8)
    env = env * (d < RADIUS).astype(jnp.float32)
    kb = basis * env
    wk = wk_ref[...]
    k0_ref[...] = jnp.dot(kb, wk[:, 0:128],
                          preferred_element_type=jnp.float32)
    k1_ref[...] = jnp.dot(kb, wk[:, 128:256],
                          preferred_element_type=jnp.float32)
    k2_ref[...] = jnp.dot(kb, wk[:, 256:384],
                          preferred_element_type=jnp.float32)
    k3_ref[...] = jnp.dot(kb, wk[:, 384:512],
                          preferred_element_type=jnp.float32)


def _sc_body(h_hbm, k_hbm, src_hbm, dst_hbm, zeros_hbm, out_hbm,
             idx_s, idx_d, rows, kbuf, acc, sem_g, sem_s):
    c = lax.axis_index("c")
    s = lax.axis_index("s")
    w = c * NS + s

    pltpu.sync_copy(zeros_hbm.at[pl.ds(s * ROWS_PER_SUB, ROWS_PER_SUB), :],
                    acc.at[pl.ds(s * ROWS_PER_SUB, ROWS_PER_SUB), :])

    @pl.when(s == NS - 1)
    def _():
        tail = NS * ROWS_PER_SUB
        pltpu.sync_copy(zeros_hbm.at[pl.ds(tail, N - tail), :],
                        acc.at[pl.ds(tail, N - tail), :])

    plsc.subcore_barrier()

    def do_chunk(j, b, first):
        base = w * EW + j * CHUNK
        if not first:
            pltpu.make_async_copy(rows[b], acc.at[idx_d[b]],
                                  sem_s[b]).wait()
        pltpu.sync_copy(src_hbm.at[pl.ds(base, CHUNK)], idx_s)
        gd = pltpu.async_copy(h_hbm.at[idx_s], rows[b], sem_g)
        pltpu.sync_copy(dst_hbm.at[pl.ds(base, CHUNK)], idx_d[b])
        pltpu.sync_copy(k_hbm.at[pl.ds(base, CHUNK), :], kbuf)
        gd.wait()

        def mul_body(i, carry2):
            for u in range(8):
                sl = pl.ds(u * 16, 16)
                rows[b][i, sl] = rows[b][i, sl] * kbuf[i, sl]
            return carry2

        lax.fori_loop(0, CHUNK, mul_body, 0)
        pltpu.async_copy(rows[b], acc.at[idx_d[b]], sem_s[b], add=True)

    def pair_body(j2, carry):
        for b in range(2):
            @pl.when(j2 > 0)
            def _():
                do_chunk(j2 * 2 + b, b, False)

            @pl.when(j2 == 0)
            def _():
                do_chunk(j2 * 2 + b, b, True)
        return carry

    lax.fori_loop(0, (NCHUNK - 1) // 2, pair_body, 0)
    do_chunk(NCHUNK - 1, 0, False)
    pltpu.make_async_copy(rows[0], acc.at[idx_d[0]], sem_s[0]).wait()
    pltpu.make_async_copy(rows[1], acc.at[idx_d[1]], sem_s[1]).wait()

    plsc.subcore_barrier()
    pltpu.sync_copy(acc.at[pl.ds(s * ROWS_PER_SUB, ROWS_PER_SUB), :],
                    out_hbm.at[c, pl.ds(s * ROWS_PER_SUB, ROWS_PER_SUB), :])

    @pl.when(s == NS - 1)
    def _():
        tail = NS * ROWS_PER_SUB
        pltpu.sync_copy(acc.at[pl.ds(tail, N - tail), :],
                        out_hbm.at[c, pl.ds(tail, N - tail), :])


def _node_body(part_ref, h_ref, batch_ref, bk_ref, g_ref, be_ref,
               w1_ref, b1_ref, w2_ref, b2_ref, ls_ref, wr_ref, br_ref,
               hout_ref, pool_ref):
    i = pl.program_id(0)
    agg = part_ref[0] + part_ref[1] + bk_ref[0, :]
    m = jnp.mean(agg, axis=-1, keepdims=True)
    ctr = agg - m
    v = jnp.mean(ctr * ctr, axis=-1, keepdims=True)
    y = ctr * jax.lax.rsqrt(v + 1e-5) * g_ref[0, :] + be_ref[0, :]
    y = _gelu(jnp.dot(y, w1_ref[...],
                      preferred_element_type=jnp.float32) + b1_ref[0, :])
    y = jnp.dot(y, w2_ref[...],
                preferred_element_type=jnp.float32) + b2_ref[0, :]
    hn = h_ref[...] + ls_ref[0, :] * y
    hout_ref[...] = hn
    r = jnp.dot(hn, wr_ref[...],
                preferred_element_type=jnp.float32) + br_ref[0, 0]
    b = batch_ref[0]
    gid = jax.lax.broadcasted_iota(jnp.int32, (GRAPHS, BN), 0)
    oh = (gid == b).astype(jnp.float32)
    pr = jnp.dot(oh, r, preferred_element_type=jnp.float32)

    @pl.when(i == 0)
    def _():
        pool_ref[...] = pr

    @pl.when(i != 0)
    def _():
        pool_ref[...] = pool_ref[...] + pr


def _const_spec(shape):
    return pl.BlockSpec(shape, lambda i: (0,) * len(shape))


@jax.jit
def kernel(x, attr, dists, edge_index, batch, params):
    src = edge_index[0]
    dst = edge_index[1]
    zeros = jnp.zeros((N, HID), jnp.float32)
    batch3 = batch.reshape(N // BN, 1, BN)

    h = pl.pallas_call(
        _embed_body,
        grid=(N // BN,),
        in_specs=[pl.BlockSpec((BN, 128), lambda i: (i, 0)),
                  _const_spec((128, HID))],
        out_specs=pl.BlockSpec((BN, HID), lambda i: (i, 0)),
        out_shape=jax.ShapeDtypeStruct((N, HID), jnp.float32),
    )(x, params['Wx'])

    wb1 = params['Wb1'][_PERM, :]
    wk = jnp.concatenate([params['Wk%d' % i] for i in range(LAYERS)], axis=1)
    kspec = pl.BlockSpec((BE, HID), lambda i: (i, 0))
    ks = pl.pallas_call(
        _edge_body,
        grid=(E // BE,),
        in_specs=[pl.BlockSpec((BE, 3), lambda i: (i, 0)),
                  pl.BlockSpec((BE, 1), lambda i: (i, 0)),
                  _const_spec((120, HID)),
                  _const_spec((1, HID)),
                  _const_spec((HID, HID)),
                  _const_spec((1, HID)),
                  _const_spec((HID, 4 * HID))],
        out_specs=[kspec, kspec, kspec, kspec],
        out_shape=[jax.ShapeDtypeStruct((E, HID), jnp.float32)] * 4,
    )(attr, dists, wb1, params['bb1'].reshape(1, HID),
      params['Wb2'], params['bb2'].reshape(1, HID), wk)

    sc_call = pl.kernel(
        _sc_body,
        out_type=jax.ShapeDtypeStruct((NC, N, HID), jnp.float32),
        mesh=plsc.VectorSubcoreMesh(core_axis_name="c", subcore_axis_name="s",
                                    num_cores=NC, num_subcores=NS),
        scratch_types=[
            pltpu.VMEM((CHUNK,), jnp.int32),
            [pltpu.VMEM((CHUNK,), jnp.int32) for _ in range(2)],
            [pltpu.VMEM((CHUNK, HID), jnp.float32) for _ in range(2)],
            pltpu.VMEM((CHUNK, HID), jnp.float32),
            pltpu.VMEM_SHARED((N, HID), jnp.float32),
            pltpu.SemaphoreType.DMA,
            [pltpu.SemaphoreType.DMA for _ in range(2)],
        ],
    )

    node_call = pl.pallas_call(
        _node_body,
        grid=(N // BN,),
        in_specs=[pl.BlockSpec((NC, BN, HID), lambda i: (0, i, 0)),
                  pl.BlockSpec((BN, HID), lambda i: (i, 0)),
                  pl.BlockSpec((1, 1, BN), lambda i: (i, 0, 0)),
                  _const_spec((1, HID)),
                  _const_spec((1, HID)),
                  _const_spec((1, HID)),
                  _const_spec((HID, 4 * HID)),
                  _const_spec((1, 4 * HID)),
                  _const_spec((4 * HID, HID)),
                  _const_spec((1, HID)),
                  _const_spec((1, HID)),
                  _const_spec((HID, 1)),
                  _const_spec((1, 1))],
        out_specs=[pl.BlockSpec((BN, HID), lambda i: (i, 0)),
                   pl.BlockSpec((GRAPHS, 1), lambda i: (0, 0))],
        out_shape=[jax.ShapeDtypeStruct((N, HID), jnp.float32),
                   jax.ShapeDtypeStruct((GRAPHS, 1), jnp.float32)],
    )

    pooled = None
    for i in range(LAYERS):
        part = sc_call(h, ks[i], src, dst, zeros)
        h, pr = node_call(
            part, h, batch3,
            params['bk%d' % i].reshape(1, HID),
            params['g%d' % i].reshape(1, HID),
            params['be%d' % i].reshape(1, HID),
            params['W1_%d' % i],
            params['b1_%d' % i].reshape(1, 4 * HID),
            params['W2_%d' % i],
            params['b2_%d' % i].reshape(1, HID),
            params['ls%d' % i].reshape(1, HID),
            params['Wr%d' % i],
            params['br%d' % i].reshape(1, 1))
        pooled = pr if pooled is None else pooled + pr

    return pooled * (1.0 / LAYERS)


# speedup vs baseline: 1.5730x; 1.1336x over previous
import functools

import jax
import jax.numpy as jnp
import numpy as np
from jax import lax
from jax.experimental import pallas as pl
from jax.experimental.pallas import tpu as pltpu
from jax.experimental.pallas import tpu_sc as plsc

N = 10000
E = 160000
HID = 128
GRAPHS = 64
LAYERS = 4
RADIUS = 1.0
CUT_P = 6.0

NC = 2
NS = 16
NW = NC * NS
EW = E // NW
CHUNK = 40
NCHUNK = EW // CHUNK
ROWS_PER_SUB = 624

BE = 1000
BN = 1000


def _gelu(v):
    c = np.sqrt(2.0 / np.pi).astype(np.float32)
    return 0.5 * v * (1.0 + jnp.tanh(c * (v + 0.044715 * (v * v * v))))


def _poly_perm():
    idx = list(range(3))
    off = 3
    for t in (2, 3, 4):
        n = 3 ** t
        for m in range(n):
            dig = []
            mm = m
            for _ in range(t):
                dig.append(mm % 3)
                mm //= 3
            rev = 0
            for d in dig:
                rev = rev * 3 + d
            idx.append(off + rev)
        off += n
    return np.array(idx, dtype=np.int32)


_PERM = _poly_perm()


def _embed_body(x_ref, wx_ref, h_ref):
    h_ref[...] = jnp.dot(x_ref[...], wx_ref[...],
                         preferred_element_type=jnp.float32)


def _edge_body(attr_ref, dists_ref, wb1_ref, bb1_ref, wb2_ref, bb2_ref,
               wk_ref, k0_ref, k1_ref, k2_ref, k3_ref):
    a = attr_ref[...]
    a0 = a[:, 0:1]
    a1 = a[:, 1:2]
    a2 = a[:, 2:3]
    f2 = jnp.concatenate([a * a0, a * a1, a * a2], axis=1)
    f3 = jnp.concatenate([f2 * a0, f2 * a1, f2 * a2], axis=1)
    f4 = jnp.concatenate([f3 * a0, f3 * a1, f3 * a2], axis=1)
    feats = jnp.concatenate([a, f2, f3, f4], axis=1)
    hb = _gelu(jnp.dot(feats, wb1_ref[...],
                       preferred_element_type=jnp.float32) + bb1_ref[0, :])
    basis = _gelu(jnp.dot(hb, wb2_ref[...],
                          preferred_element_type=jnp.float32) + bb2_ref[0, :])
    d = dists_ref[...]
    p = CUT_P
    r = d * (1.0 / RADIUS)
    r2 = r * r
    r3 = r2 * r
    r6 = r3 * r3
    r7 = r6 * r
    r8 = r7 * r
    env = (1.0 - ((p + 1.0) * (p + 2.0) / 2.0) * r6
           + p * (p + 2.0) * r7
           - (p * (p + 1.0) / 2.0) * r8)
    env = env * (d < RADIUS).astype(jnp.float32)
    kb = basis * env
    wk = wk_ref[...]
    k0_ref[...] = jnp.dot(kb, wk[:, 0:128],
                          preferred_element_type=jnp.float32)
    k1_ref[...] = jnp.dot(kb, wk[:, 128:256],
                          preferred_element_type=jnp.float32)
    k2_ref[...] = jnp.dot(kb, wk[:, 256:384],
                          preferred_element_type=jnp.float32)
    k3_ref[...] = jnp.dot(kb, wk[:, 384:512],
                          preferred_element_type=jnp.float32)


def _sc_body(h_hbm, k_hbm, src_hbm, dst_hbm, zeros_hbm, out_hbm,
             idx_s, idx_d, rows, kbuf, acc, sem_g, sem_s, sem_i):
    c = lax.axis_index("c")
    s = lax.axis_index("s")
    w = c * NS + s

    pltpu.sync_copy(zeros_hbm.at[pl.ds(s * ROWS_PER_SUB, ROWS_PER_SUB), :],
                    acc.at[pl.ds(s * ROWS_PER_SUB, ROWS_PER_SUB), :])

    @pl.when(s == NS - 1)
    def _():
        tail = NS * ROWS_PER_SUB
        pltpu.sync_copy(zeros_hbm.at[pl.ds(tail, N - tail), :],
                        acc.at[pl.ds(tail, N - tail), :])

    plsc.subcore_barrier()

    def do_chunk(j, b, first, last):
        base = w * EW + j * CHUNK
        if not first:
            pltpu.make_async_copy(rows[b], acc.at[idx_d[b]],
                                  sem_s[b]).wait()
        pltpu.make_async_copy(src_hbm.at[pl.ds(base, CHUNK)],
                              idx_s[b], sem_i[b]).wait()
        gd = pltpu.async_copy(h_hbm.at[idx_s[b]], rows[b], sem_g)
        if not last:
            nbase = w * EW + (j + 1) * CHUNK
            pltpu.async_copy(src_hbm.at[pl.ds(nbase, CHUNK)],
                             idx_s[1 - b], sem_i[1 - b])
        pltpu.sync_copy(dst_hbm.at[pl.ds(base, CHUNK)], idx_d[b])
        pltpu.sync_copy(k_hbm.at[pl.ds(base, CHUNK), :], kbuf)
        gd.wait()

        def mul_body(i, carry2):
            for u in range(8):
                sl = pl.ds(u * 16, 16)
                rows[b][i, sl] = rows[b][i, sl] * kbuf[i, sl]
            return carry2

        lax.fori_loop(0, CHUNK, mul_body, 0)
        pltpu.async_copy(rows[b], acc.at[idx_d[b]], sem_s[b], add=True)

    def pair_body(j2, carry):
        for b in range(2):
            @pl.when(j2 > 0)
            def _():
                do_chunk(j2 * 2 + b, b, False, False)

            @pl.when(j2 == 0)
            def _():
                do_chunk(j2 * 2 + b, b, True, False)
        return carry

    pltpu.async_copy(src_hbm.at[pl.ds(w * EW, CHUNK)], idx_s[0], sem_i[0])
    lax.fori_loop(0, (NCHUNK - 1) // 2, pair_body, 0)
    do_chunk(NCHUNK - 1, 0, False, True)
    pltpu.make_async_copy(rows[0], acc.at[idx_d[0]], sem_s[0]).wait()
    pltpu.make_async_copy(rows[1], acc.at[idx_d[1]], sem_s[1]).wait()

    plsc.subcore_barrier()
    pltpu.sync_copy(acc.at[pl.ds(s * ROWS_PER_SUB, ROWS_PER_SUB), :],
                    out_hbm.at[c, pl.ds(s * ROWS_PER_SUB, ROWS_PER_SUB), :])

    @pl.when(s == NS - 1)
    def _():
        tail = NS * ROWS_PER_SUB
        pltpu.sync_copy(acc.at[pl.ds(tail, N - tail), :],
                        out_hbm.at[c, pl.ds(tail, N - tail), :])


def _node_body(part_ref, h_ref, batch_ref, bk_ref, g_ref, be_ref,
               w1_ref, b1_ref, w2_ref, b2_ref, ls_ref, wr_ref, br_ref,
               hout_ref, pool_ref):
    i = pl.program_id(0)
    agg = part_ref[0] + part_ref[1] + bk_ref[0, :]
    m = jnp.mean(agg, axis=-1, keepdims=True)
    ctr = agg - m
    v = jnp.mean(ctr * ctr, axis=-1, keepdims=True)
    y = ctr * jax.lax.rsqrt(v + 1e-5) * g_ref[0, :] + be_ref[0, :]
    y = _gelu(jnp.dot(y, w1_ref[...],
                      preferred_element_type=jnp.float32) + b1_ref[0, :])
    y = jnp.dot(y, w2_ref[...],
                preferred_element_type=jnp.float32) + b2_ref[0, :]
    hn = h_ref[...] + ls_ref[0, :] * y
    hout_ref[...] = hn
    r = jnp.dot(hn, wr_ref[...],
                preferred_element_type=jnp.float32) + br_ref[0, 0]
    b = batch_ref[0]
    gid = jax.lax.broadcasted_iota(jnp.int32, (GRAPHS, BN), 0)
    oh = (gid == b).astype(jnp.float32)
    pr = jnp.dot(oh, r, preferred_element_type=jnp.float32)

    @pl.when(i == 0)
    def _():
        pool_ref[...] = pr

    @pl.when(i != 0)
    def _():
        pool_ref[...] = pool_ref[...] + pr


def _const_spec(shape):
    return pl.BlockSpec(shape, lambda i: (0,) * len(shape))


@jax.jit
def kernel(x, attr, dists, edge_index, batch, params):
    src = edge_index[0]
    dst = edge_index[1]
    zeros = jnp.zeros((N, HID), jnp.float32)
    batch3 = batch.reshape(N // BN, 1, BN)

    h = pl.pallas_call(
        _embed_body,
        grid=(N // BN,),
        in_specs=[pl.BlockSpec((BN, 128), lambda i: (i, 0)),
                  _const_spec((128, HID))],
        out_specs=pl.BlockSpec((BN, HID), lambda i: (i, 0)),
        out_shape=jax.ShapeDtypeStruct((N, HID), jnp.float32),
    )(x, params['Wx'])

    wb1 = params['Wb1'][_PERM, :]
    wk = jnp.concatenate([params['Wk%d' % i] for i in range(LAYERS)], axis=1)
    kspec = pl.BlockSpec((BE, HID), lambda i: (i, 0))
    ks = pl.pallas_call(
        _edge_body,
        grid=(E // BE,),
        in_specs=[pl.BlockSpec((BE, 3), lambda i: (i, 0)),
                  pl.BlockSpec((BE, 1), lambda i: (i, 0)),
                  _const_spec((120, HID)),
                  _const_spec((1, HID)),
                  _const_spec((HID, HID)),
                  _const_spec((1, HID)),
                  _const_spec((HID, 4 * HID))],
        out_specs=[kspec, kspec, kspec, kspec],
        out_shape=[jax.ShapeDtypeStruct((E, HID), jnp.float32)] * 4,
    )(attr, dists, wb1, params['bb1'].reshape(1, HID),
      params['Wb2'], params['bb2'].reshape(1, HID), wk)

    sc_call = pl.kernel(
        _sc_body,
        out_type=jax.ShapeDtypeStruct((NC, N, HID), jnp.float32),
        mesh=plsc.VectorSubcoreMesh(core_axis_name="c", subcore_axis_name="s",
                                    num_cores=NC, num_subcores=NS),
        scratch_types=[
            [pltpu.VMEM((CHUNK,), jnp.int32) for _ in range(2)],
            [pltpu.VMEM((CHUNK,), jnp.int32) for _ in range(2)],
            [pltpu.VMEM((CHUNK, HID), jnp.float32) for _ in range(2)],
            pltpu.VMEM((CHUNK, HID), jnp.float32),
            pltpu.VMEM_SHARED((N, HID), jnp.float32),
            pltpu.SemaphoreType.DMA,
            [pltpu.SemaphoreType.DMA for _ in range(2)],
            [pltpu.SemaphoreType.DMA for _ in range(2)],
        ],
    )

    node_call = pl.pallas_call(
        _node_body,
        grid=(N // BN,),
        in_specs=[pl.BlockSpec((NC, BN, HID), lambda i: (0, i, 0)),
                  pl.BlockSpec((BN, HID), lambda i: (i, 0)),
                  pl.BlockSpec((1, 1, BN), lambda i: (i, 0, 0)),
                  _const_spec((1, HID)),
                  _const_spec((1, HID)),
                  _const_spec((1, HID)),
                  _const_spec((HID, 4 * HID)),
                  _const_spec((1, 4 * HID)),
                  _const_spec((4 * HID, HID)),
                  _const_spec((1, HID)),
                  _const_spec((1, HID)),
                  _const_spec((HID, 1)),
                  _const_spec((1, 1))],
        out_specs=[pl.BlockSpec((BN, HID), lambda i: (i, 0)),
                   pl.BlockSpec((GRAPHS, 1), lambda i: (0, 0))],
        out_shape=[jax.ShapeDtypeStruct((N, HID), jnp.float32),
                   jax.ShapeDtypeStruct((GRAPHS, 1), jnp.float32)],
    )

    pooled = None
    for i in range(LAYERS):
        part = sc_call(h, ks[i], src, dst, zeros)
        h, pr = node_call(
            part, h, batch3,
            params['bk%d' % i].reshape(1, HID),
            params['g%d' % i].reshape(1, HID),
            params['be%d' % i].reshape(1, HID),
            params['W1_%d' % i],
            params['b1_%d' % i].reshape(1, 4 * HID),
            params['W2_%d' % i],
            params['b2_%d' % i].reshape(1, HID),
            params['ls%d' % i].reshape(1, HID),
            params['Wr%d' % i],
            params['br%d' % i].reshape(1, 1))
        pooled = pr if pooled is None else pooled + pr

    return pooled * (1.0 / LAYERS)


# speedup vs baseline: 1.7699x; 1.1252x over previous
import functools

import jax
import jax.numpy as jnp
import numpy as np
from jax import lax
from jax.experimental import pallas as pl
from jax.experimental.pallas import tpu as pltpu
from jax.experimental.pallas import tpu_sc as plsc

N = 10000
E = 160000
HID = 128
GRAPHS = 64
LAYERS = 4
RADIUS = 1.0
CUT_P = 6.0

NC = 2
NS = 16
NW = NC * NS
EW = E // NW
CHUNK = 40
NCHUNK = EW // CHUNK
ROWS_PER_SUB = 624

BE = 1000
BN = 1000


def _gelu(v):
    c = np.sqrt(2.0 / np.pi).astype(np.float32)
    return 0.5 * v * (1.0 + jnp.tanh(c * (v + 0.044715 * (v * v * v))))


def _poly_perm():
    idx = list(range(3))
    off = 3
    for t in (2, 3, 4):
        n = 3 ** t
        for m in range(n):
            dig = []
            mm = m
            for _ in range(t):
                dig.append(mm % 3)
                mm //= 3
            rev = 0
            for d in dig:
                rev = rev * 3 + d
            idx.append(off + rev)
        off += n
    return np.array(idx, dtype=np.int32)


_PERM = _poly_perm()


def _embed_body(x_ref, wx_ref, h_ref):
    h_ref[...] = jnp.dot(x_ref[...], wx_ref[...],
                         preferred_element_type=jnp.float32)


def _edge_body(attr_ref, dists_ref, wb1_ref, bb1_ref, wb2_ref, bb2_ref,
               wk_ref, k0_ref, k1_ref, k2_ref, k3_ref):
    a = attr_ref[...]
    a0 = a[:, 0:1]
    a1 = a[:, 1:2]
    a2 = a[:, 2:3]
    f2 = jnp.concatenate([a * a0, a * a1, a * a2], axis=1)
    f3 = jnp.concatenate([f2 * a0, f2 * a1, f2 * a2], axis=1)
    f4 = jnp.concatenate([f3 * a0, f3 * a1, f3 * a2], axis=1)
    feats = jnp.concatenate([a, f2, f3, f4], axis=1)
    hb = _gelu(jnp.dot(feats, wb1_ref[...],
                       preferred_element_type=jnp.float32) + bb1_ref[0, :])
    basis = _gelu(jnp.dot(hb, wb2_ref[...],
                          preferred_element_type=jnp.float32) + bb2_ref[0, :])
    d = dists_ref[...]
    p = CUT_P
    r = d * (1.0 / RADIUS)
    r2 = r * r
    r3 = r2 * r
    r6 = r3 * r3
    r7 = r6 * r
    r8 = r7 * r
    env = (1.0 - ((p + 1.0) * (p + 2.0) / 2.0) * r6
           + p * (p + 2.0) * r7
           - (p * (p + 1.0) / 2.0) * r8)
    env = env * (d < RADIUS).astype(jnp.float32)
    kb = basis * env
    wk = wk_ref[...]
    k0_ref[...] = jnp.dot(kb, wk[:, 0:128],
                          preferred_element_type=jnp.float32)
    k1_ref[...] = jnp.dot(kb, wk[:, 128:256],
                          preferred_element_type=jnp.float32)
    k2_ref[...] = jnp.dot(kb, wk[:, 256:384],
                          preferred_element_type=jnp.float32)
    k3_ref[...] = jnp.dot(kb, wk[:, 384:512],
                          preferred_element_type=jnp.float32)


def _sc_body(h_hbm, k_hbm, src_hbm, dst_hbm, zeros_hbm, out_hbm,
             idx_s, idx_d, rows, kbuf, acc, sem_g, sem_s, sem_i, sem_k):
    c = lax.axis_index("c")
    s = lax.axis_index("s")
    w = c * NS + s

    pltpu.sync_copy(zeros_hbm.at[pl.ds(s * ROWS_PER_SUB, ROWS_PER_SUB), :],
                    acc.at[pl.ds(s * ROWS_PER_SUB, ROWS_PER_SUB), :])

    @pl.when(s == NS - 1)
    def _():
        tail = NS * ROWS_PER_SUB
        pltpu.sync_copy(zeros_hbm.at[pl.ds(tail, N - tail), :],
                        acc.at[pl.ds(tail, N - tail), :])

    plsc.subcore_barrier()

    def do_chunk(j, b, first, last):
        base = w * EW + j * CHUNK
        if not first:
            pltpu.make_async_copy(rows[b], acc.at[idx_d[b]],
                                  sem_s[b]).wait()
        pltpu.make_async_copy(src_hbm.at[pl.ds(base, CHUNK)],
                              idx_s[b], sem_i[b]).wait()
        gd = pltpu.async_copy(h_hbm.at[idx_s[b]], rows[b], sem_g)
        if not last:
            nbase = w * EW + (j + 1) * CHUNK
            pltpu.async_copy(src_hbm.at[pl.ds(nbase, CHUNK)],
                             idx_s[1 - b], sem_i[1 - b])
            pltpu.async_copy(k_hbm.at[pl.ds(nbase, CHUNK), :],
                             kbuf[1 - b], sem_k[1 - b])
        pltpu.sync_copy(dst_hbm.at[pl.ds(base, CHUNK)], idx_d[b])
        pltpu.make_async_copy(k_hbm.at[pl.ds(base, CHUNK), :],
                              kbuf[b], sem_k[b]).wait()
        gd.wait()

        def mul_body(i, carry2):
            for u in range(8):
                sl = pl.ds(u * 16, 16)
                rows[b][i, sl] = rows[b][i, sl] * kbuf[b][i, sl]
            return carry2

        lax.fori_loop(0, CHUNK, mul_body, 0)
        pltpu.async_copy(rows[b], acc.at[idx_d[b]], sem_s[b], add=True)

    def pair_body(j2, carry):
        for b in range(2):
            @pl.when(j2 > 0)
            def _():
                do_chunk(j2 * 2 + b, b, False, False)

            @pl.when(j2 == 0)
            def _():
                do_chunk(j2 * 2 + b, b, True, False)
        return carry

    pltpu.async_copy(src_hbm.at[pl.ds(w * EW, CHUNK)], idx_s[0], sem_i[0])
    pltpu.async_copy(k_hbm.at[pl.ds(w * EW, CHUNK), :], kbuf[0], sem_k[0])
    lax.fori_loop(0, (NCHUNK - 1) // 2, pair_body, 0)
    do_chunk(NCHUNK - 1, 0, False, True)
    pltpu.make_async_copy(rows[0], acc.at[idx_d[0]], sem_s[0]).wait()
    pltpu.make_async_copy(rows[1], acc.at[idx_d[1]], sem_s[1]).wait()

    plsc.subcore_barrier()
    pltpu.sync_copy(acc.at[pl.ds(s * ROWS_PER_SUB, ROWS_PER_SUB), :],
                    out_hbm.at[c, pl.ds(s * ROWS_PER_SUB, ROWS_PER_SUB), :])

    @pl.when(s == NS - 1)
    def _():
        tail = NS * ROWS_PER_SUB
        pltpu.sync_copy(acc.at[pl.ds(tail, N - tail), :],
                        out_hbm.at[c, pl.ds(tail, N - tail), :])


def _node_body(part_ref, h_ref, batch_ref, bk_ref, g_ref, be_ref,
               w1_ref, b1_ref, w2_ref, b2_ref, ls_ref, wr_ref, br_ref,
               hout_ref, pool_ref):
    i = pl.program_id(0)
    agg = part_ref[0] + part_ref[1] + bk_ref[0, :]
    m = jnp.mean(agg, axis=-1, keepdims=True)
    ctr = agg - m
    v = jnp.mean(ctr * ctr, axis=-1, keepdims=True)
    y = ctr * jax.lax.rsqrt(v + 1e-5) * g_ref[0, :] + be_ref[0, :]
    y = _gelu(jnp.dot(y, w1_ref[...],
                      preferred_element_type=jnp.float32) + b1_ref[0, :])
    y = jnp.dot(y, w2_ref[...],
                preferred_element_type=jnp.float32) + b2_ref[0, :]
    hn = h_ref[...] + ls_ref[0, :] * y
    hout_ref[...] = hn
    r = jnp.dot(hn, wr_ref[...],
                preferred_element_type=jnp.float32) + br_ref[0, 0]
    b = batch_ref[0]
    gid = jax.lax.broadcasted_iota(jnp.int32, (GRAPHS, BN), 0)
    oh = (gid == b).astype(jnp.float32)
    pr = jnp.dot(oh, r, preferred_element_type=jnp.float32)

    @pl.when(i == 0)
    def _():
        pool_ref[...] = pr

    @pl.when(i != 0)
    def _():
        pool_ref[...] = pool_ref[...] + pr


def _const_spec(shape):
    return pl.BlockSpec(shape, lambda i: (0,) * len(shape))


@jax.jit
def kernel(x, attr, dists, edge_index, batch, params):
    src = edge_index[0]
    dst = edge_index[1]
    zeros = jnp.zeros((N, HID), jnp.float32)
    batch3 = batch.reshape(N // BN, 1, BN)

    h = pl.pallas_call(
        _embed_body,
        grid=(N // BN,),
        in_specs=[pl.BlockSpec((BN, 128), lambda i: (i, 0)),
                  _const_spec((128, HID))],
        out_specs=pl.BlockSpec((BN, HID), lambda i: (i, 0)),
        out_shape=jax.ShapeDtypeStruct((N, HID), jnp.float32),
    )(x, params['Wx'])

    wb1 = params['Wb1'][_PERM, :]
    wk = jnp.concatenate([params['Wk%d' % i] for i in range(LAYERS)], axis=1)
    kspec = pl.BlockSpec((BE, HID), lambda i: (i, 0))
    ks = pl.pallas_call(
        _edge_body,
        grid=(E // BE,),
        in_specs=[pl.BlockSpec((BE, 3), lambda i: (i, 0)),
                  pl.BlockSpec((BE, 1), lambda i: (i, 0)),
                  _const_spec((120, HID)),
                  _const_spec((1, HID)),
                  _const_spec((HID, HID)),
                  _const_spec((1, HID)),
                  _const_spec((HID, 4 * HID))],
        out_specs=[kspec, kspec, kspec, kspec],
        out_shape=[jax.ShapeDtypeStruct((E, HID), jnp.float32)] * 4,
    )(attr, dists, wb1, params['bb1'].reshape(1, HID),
      params['Wb2'], params['bb2'].reshape(1, HID), wk)

    sc_call = pl.kernel(
        _sc_body,
        out_type=jax.ShapeDtypeStruct((NC, N, HID), jnp.float32),
        mesh=plsc.VectorSubcoreMesh(core_axis_name="c", subcore_axis_name="s",
                                    num_cores=NC, num_subcores=NS),
        scratch_types=[
            [pltpu.VMEM((CHUNK,), jnp.int32) for _ in range(2)],
            [pltpu.VMEM((CHUNK,), jnp.int32) for _ in range(2)],
            [pltpu.VMEM((CHUNK, HID), jnp.float32) for _ in range(2)],
            [pltpu.VMEM((CHUNK, HID), jnp.float32) for _ in range(2)],
            pltpu.VMEM_SHARED((N, HID), jnp.float32),
            pltpu.SemaphoreType.DMA,
            [pltpu.SemaphoreType.DMA for _ in range(2)],
            [pltpu.SemaphoreType.DMA for _ in range(2)],
            [pltpu.SemaphoreType.DMA for _ in range(2)],
        ],
    )

    node_call = pl.pallas_call(
        _node_body,
        grid=(N // BN,),
        in_specs=[pl.BlockSpec((NC, BN, HID), lambda i: (0, i, 0)),
                  pl.BlockSpec((BN, HID), lambda i: (i, 0)),
                  pl.BlockSpec((1, 1, BN), lambda i: (i, 0, 0)),
                  _const_spec((1, HID)),
                  _const_spec((1, HID)),
                  _const_spec((1, HID)),
                  _const_spec((HID, 4 * HID)),
                  _const_spec((1, 4 * HID)),
                  _const_spec((4 * HID, HID)),
                  _const_spec((1, HID)),
                  _const_spec((1, HID)),
                  _const_spec((HID, 1)),
                  _const_spec((1, 1))],
        out_specs=[pl.BlockSpec((BN, HID), lambda i: (i, 0)),
                   pl.BlockSpec((GRAPHS, 1), lambda i: (0, 0))],
        out_shape=[jax.ShapeDtypeStruct((N, HID), jnp.float32),
                   jax.ShapeDtypeStruct((GRAPHS, 1), jnp.float32)],
    )

    pooled = None
    for i in range(LAYERS):
        part = sc_call(h, ks[i], src, dst, zeros)
        h, pr = node_call(
            part, h, batch3,
            params['bk%d' % i].reshape(1, HID),
            params['g%d' % i].reshape(1, HID),
            params['be%d' % i].reshape(1, HID),
            params['W1_%d' % i],
            params['b1_%d' % i].reshape(1, 4 * HID),
            params['W2_%d' % i],
            params['b2_%d' % i].reshape(1, HID),
            params['ls%d' % i].reshape(1, HID),
            params['Wr%d' % i],
            params['br%d' % i].reshape(1, 1))
        pooled = pr if pooled is None else pooled + pr

    return pooled * (1.0 / LAYERS)
